# baseline (speedup 1.0000x reference)
import functools

import jax
import jax.numpy as jnp
from jax import lax
from jax.experimental import pallas as pl
from jax.experimental.pallas import tpu as pltpu
from jax.experimental.pallas import tpu_sc as plsc

B = 1024
N = 20000
N_DN = 1300
N_MBON = 96
L = 16
DN_PAD = 1312
MBON_PAD = 96
NC = 2
NS = 16
NW = NC * NS
ROWS_PER_W = B // NW


def _sc_body(v_hbm, dn_idx_hbm, mbon_idx_hbm, dn_out_hbm, mbon_out_hbm,
             dn_idx_v, mbon_idx_v, row_v, dn_row, mbon_row):
    wid = lax.axis_index("s") * NC + lax.axis_index("c")
    base = wid * ROWS_PER_W

    pltpu.sync_copy(dn_idx_hbm, dn_idx_v)
    pltpu.sync_copy(mbon_idx_hbm, mbon_idx_v)

    def row_body(r, carry):
        pltpu.sync_copy(v_hbm.at[base + r], row_v)
        for c in range(DN_PAD // L):
            ii = dn_idx_v[pl.ds(c * L, L)]
            dn_row[pl.ds(c * L, L)] = plsc.load_gather(row_v, [ii])
        for c in range(MBON_PAD // L):
            ii = mbon_idx_v[pl.ds(c * L, L)]
            mbon_row[pl.ds(c * L, L)] = plsc.load_gather(row_v, [ii])
        pltpu.sync_copy(dn_row.at[pl.ds(0, N_DN)], dn_out_hbm.at[base + r])
        pltpu.sync_copy(mbon_row, mbon_out_hbm.at[base + r])
        return carry

    lax.fori_loop(0, ROWS_PER_W, row_body, 0)


@jax.jit
def kernel(v, dn_id, mbon_id):
    dn_idx = jnp.concatenate(
        [dn_id.astype(jnp.int32),
         jnp.zeros((DN_PAD - N_DN,), jnp.int32)])
    mbon_idx = mbon_id.astype(jnp.int32)

    mesh = plsc.VectorSubcoreMesh(core_axis_name="c", subcore_axis_name="s")
    run = pl.kernel(
        _sc_body,
        mesh=mesh,
        compiler_params=pltpu.CompilerParams(needs_layout_passes=False,
                                             use_tc_tiling_on_sc=False),
        out_type=(jax.ShapeDtypeStruct((B, N_DN), jnp.float32),
                  jax.ShapeDtypeStruct((B, N_MBON), jnp.float32)),
        scratch_types=[
            pltpu.VMEM((DN_PAD,), jnp.int32),
            pltpu.VMEM((MBON_PAD,), jnp.int32),
            pltpu.VMEM((N,), jnp.float32),
            pltpu.VMEM((DN_PAD,), jnp.float32),
            pltpu.VMEM((MBON_PAD,), jnp.float32),
        ],
    )
    return run(v, dn_idx, mbon_idx)


# speedup vs baseline: 1.1194x; 1.1194x over previous
import functools

import jax
import jax.numpy as jnp
from jax import lax
from jax.experimental import pallas as pl
from jax.experimental.pallas import tpu as pltpu
from jax.experimental.pallas import tpu_sc as plsc

B = 1024
N = 20000
N_DN = 1300
N_MBON = 96
L = 16
DN_PAD = 1312
MBON_PAD = 96
NC = 2
NS = 16
NW = NC * NS
ROWS_PER_W = B // NW


def _sc_body(v_hbm, dn_idx_hbm, mbon_idx_hbm, dn_out_hbm, mbon_out_hbm,
             dn_idx_v, mbon_idx_v,
             rv0, rv1, dr0, dr1, mb0, mb1,
             si0, si1, sd0, sd1, sm0, sm1):
    wid = lax.axis_index("s") * NC + lax.axis_index("c")
    base = wid * ROWS_PER_W
    rv = (rv0, rv1)
    dr = (dr0, dr1)
    mb = (mb0, mb1)
    si = (si0, si1)
    sd = (sd0, sd1)
    sm = (sm0, sm1)

    pltpu.sync_copy(dn_idx_hbm, dn_idx_v)
    pltpu.sync_copy(mbon_idx_hbm, mbon_idx_v)

    for b in range(2):
        pltpu.async_copy(v_hbm.at[base + b], rv[b], si[b])

    def outer(g, carry):
        for b in range(2):
            r = 2 * g + b
            pltpu.make_async_copy(v_hbm.at[base + r], rv[b], si[b]).wait()

            @pl.when(g > 0)
            def _():
                pltpu.make_async_copy(dr[b].at[pl.ds(0, N_DN)],
                                      dn_out_hbm.at[base + r], sd[b]).wait()
                pltpu.make_async_copy(mb[b], mbon_out_hbm.at[base + r],
                                      sm[b]).wait()

            for c in range(DN_PAD // L):
                ii = dn_idx_v[pl.ds(c * L, L)]
                dr[b][pl.ds(c * L, L)] = plsc.load_gather(rv[b], [ii])
            for c in range(MBON_PAD // L):
                ii = mbon_idx_v[pl.ds(c * L, L)]
                mb[b][pl.ds(c * L, L)] = plsc.load_gather(rv[b], [ii])

            @pl.when(g < ROWS_PER_W // 2 - 1)
            def _():
                pltpu.async_copy(v_hbm.at[base + r + 2], rv[b], si[b])

            pltpu.async_copy(dr[b].at[pl.ds(0, N_DN)],
                             dn_out_hbm.at[base + r], sd[b])
            pltpu.async_copy(mb[b], mbon_out_hbm.at[base + r], sm[b])
        return carry

    lax.fori_loop(0, ROWS_PER_W // 2, outer, 0)

    for b in range(2):
        r = ROWS_PER_W - 2 + b
        pltpu.make_async_copy(dr[b].at[pl.ds(0, N_DN)],
                              dn_out_hbm.at[base + r], sd[b]).wait()
        pltpu.make_async_copy(mb[b], mbon_out_hbm.at[base + r], sm[b]).wait()


@jax.jit
def kernel(v, dn_id, mbon_id):
    dn_idx = jnp.concatenate(
        [dn_id.astype(jnp.int32),
         jnp.zeros((DN_PAD - N_DN,), jnp.int32)])
    mbon_idx = mbon_id.astype(jnp.int32)

    mesh = plsc.VectorSubcoreMesh(core_axis_name="c", subcore_axis_name="s")
    run = pl.kernel(
        _sc_body,
        mesh=mesh,
        compiler_params=pltpu.CompilerParams(needs_layout_passes=False,
                                             use_tc_tiling_on_sc=False),
        out_type=(jax.ShapeDtypeStruct((B, N_DN), jnp.float32),
                  jax.ShapeDtypeStruct((B, N_MBON), jnp.float32)),
        scratch_types=[
            pltpu.VMEM((DN_PAD,), jnp.int32),
            pltpu.VMEM((MBON_PAD,), jnp.int32),
            pltpu.VMEM((N,), jnp.float32),
            pltpu.VMEM((N,), jnp.float32),
            pltpu.VMEM((DN_PAD,), jnp.float32),
            pltpu.VMEM((DN_PAD,), jnp.float32),
            pltpu.VMEM((MBON_PAD,), jnp.float32),
            pltpu.VMEM((MBON_PAD,), jnp.float32),
            pltpu.SemaphoreType.DMA,
            pltpu.SemaphoreType.DMA,
            pltpu.SemaphoreType.DMA,
            pltpu.SemaphoreType.DMA,
            pltpu.SemaphoreType.DMA,
            pltpu.SemaphoreType.DMA,
        ],
    )
    return run(v, dn_idx, mbon_idx)
